# Initial kernel scaffold; baseline (speedup 1.0000x reference)
import jax
import jax.numpy as jnp
from jax.experimental import pallas as pl


def kernel(x):
    raise NotImplementedError("write your pallas kernel here")


# speedup vs baseline: 14.0272x; 14.0272x over previous
import functools
import jax
import jax.numpy as jnp
from jax import lax
from jax.experimental import pallas as pl
from jax.experimental.pallas import tpu as pltpu
from jax.experimental.pallas import tpu_sc as plsc

CACHELINE = 64
THRESHOLD = 0.05
_NC = 2
_NS = 16
_NW = _NC * _NS
_L = 16
GROUP_ELEMS = CACHELINE * _L


def _make_cluster_call(num_groups: int):
    groups_per_worker = num_groups // _NW
    mesh = plsc.VectorSubcoreMesh(core_axis_name="c", subcore_axis_name="s")

    @functools.partial(
        pl.kernel,
        out_type=jax.ShapeDtypeStruct((num_groups * GROUP_ELEMS,), jnp.float32),
        mesh=mesh,
        scratch_types=[
            pltpu.VMEM((GROUP_ELEMS,), jnp.float32),
            pltpu.VMEM((GROUP_ELEMS,), jnp.float32),
        ],
    )
    def cluster(x_hbm, out_hbm, xb, bv):
        wid = lax.axis_index("s") * _NC + lax.axis_index("c")

        def group_body(g, carry):
            gi = wid * groups_per_worker + g
            base = gi * GROUP_ELEMS
            pltpu.sync_copy(x_hbm.at[pl.ds(base, GROUP_ELEMS)], xb)

            bv[pl.ds((CACHELINE - 1) * _L, _L)] = xb[pl.ds(0, _L)]

            def j_body(j, carry2):
                xj = xb[pl.ds(j * _L, _L)]

                def k_body(k, res):
                    bvk = bv[pl.ds(k * _L, _L)]
                    return jnp.where(jnp.abs(bvk - xj) < THRESHOLD, bvk, res)

                res = lax.fori_loop(CACHELINE - j, CACHELINE, k_body, xj)
                bv[pl.ds((CACHELINE - 1 - j) * _L, _L)] = jnp.where(
                    res != xj, jnp.inf, xj
                )
                xb[pl.ds(j * _L, _L)] = res
                return carry2

            lax.fori_loop(1, CACHELINE, j_body, 0)
            pltpu.sync_copy(xb, out_hbm.at[pl.ds(base, GROUP_ELEMS)])
            return carry

        lax.fori_loop(0, groups_per_worker, group_body, 0)

    return cluster


def kernel(x):
    shape = x.shape
    flat = x.reshape(-1)
    n = flat.shape[0]
    m = n // CACHELINE
    body = flat[: m * CACHELINE]

    num_groups = -(-m // _L)
    total_groups = num_groups + ((-num_groups) % _NW)
    pad_elems = total_groups * GROUP_ELEMS - m * CACHELINE
    arr = body
    if pad_elems:
        arr = jnp.concatenate([arr, jnp.zeros((pad_elems,), jnp.float32)])
    arr = arr.reshape(total_groups, _L, CACHELINE).transpose(0, 2, 1)
    arr = arr.reshape(-1)

    call = _make_cluster_call(total_groups)
    out = call(arr).reshape(total_groups, CACHELINE, _L).transpose(0, 2, 1)
    out = out.reshape(-1)[: m * CACHELINE]

    if m * CACHELINE != n:
        out = jnp.concatenate([out, flat[m * CACHELINE:]])
    return out.reshape(shape)


# speedup vs baseline: 21.9997x; 1.5684x over previous
import functools
import jax
import jax.numpy as jnp
from jax import lax
from jax.experimental import pallas as pl
from jax.experimental.pallas import tpu as pltpu
from jax.experimental.pallas import tpu_sc as plsc

CACHELINE = 64
THRESHOLD = 0.05
_NC = 2
_NS = 16
_NW = _NC * _NS
_L = 16
GROUP_ELEMS = CACHELINE * _L


def _make_cluster_call(num_groups: int):
    groups_per_worker = num_groups // _NW
    mesh = plsc.VectorSubcoreMesh(core_axis_name="c", subcore_axis_name="s")

    @functools.partial(
        pl.kernel,
        out_type=jax.ShapeDtypeStruct((num_groups * GROUP_ELEMS,), jnp.float32),
        mesh=mesh,
        scratch_types=[
            pltpu.VMEM((GROUP_ELEMS,), jnp.float32),
            pltpu.VMEM((GROUP_ELEMS,), jnp.float32),
        ],
    )
    def cluster(x_hbm, out_hbm, xb, bv):
        wid = lax.axis_index("s") * _NC + lax.axis_index("c")

        def group_body(g, carry):
            gi = wid * groups_per_worker + g
            base = gi * GROUP_ELEMS
            pltpu.sync_copy(x_hbm.at[pl.ds(base, GROUP_ELEMS)], xb)

            inf_row = jnp.full((_L,), jnp.inf, jnp.float32)

            def init_body(r, c):
                bv[pl.ds(r * _L, _L)] = inf_row
                return c

            lax.fori_loop(0, CACHELINE - 1, init_body, 0)
            bv[pl.ds((CACHELINE - 1) * _L, _L)] = xb[pl.ds(0, _L)]

            def j_body(j, carry2):
                xj = xb[pl.ds(j * _L, _L)]
                s4 = ((CACHELINE - j) // 4) * 4
                n4 = (CACHELINE - s4) // 4

                def k_body(t, res):
                    rb = (s4 + t * 4) * _L
                    for u in range(4):
                        bvk = bv[pl.ds(rb + u * _L, _L)]
                        res = jnp.where(jnp.abs(bvk - xj) < THRESHOLD, bvk, res)
                    return res

                res = lax.fori_loop(0, n4, k_body, xj)
                bv[pl.ds((CACHELINE - 1 - j) * _L, _L)] = jnp.where(
                    res != xj, jnp.inf, xj
                )
                xb[pl.ds(j * _L, _L)] = res
                return carry2

            lax.fori_loop(1, CACHELINE, j_body, 0)
            pltpu.sync_copy(xb, out_hbm.at[pl.ds(base, GROUP_ELEMS)])
            return carry

        lax.fori_loop(0, groups_per_worker, group_body, 0)

    return cluster


def kernel(x):
    shape = x.shape
    flat = x.reshape(-1)
    n = flat.shape[0]
    m = n // CACHELINE
    body = flat[: m * CACHELINE]

    num_groups = -(-m // _L)
    total_groups = num_groups + ((-num_groups) % _NW)
    pad_elems = total_groups * GROUP_ELEMS - m * CACHELINE
    arr = body
    if pad_elems:
        arr = jnp.concatenate([arr, jnp.zeros((pad_elems,), jnp.float32)])
    arr = arr.reshape(total_groups, _L, CACHELINE).transpose(0, 2, 1)
    arr = arr.reshape(-1)

    call = _make_cluster_call(total_groups)
    out = call(arr).reshape(total_groups, CACHELINE, _L).transpose(0, 2, 1)
    out = out.reshape(-1)[: m * CACHELINE]

    if m * CACHELINE != n:
        out = jnp.concatenate([out, flat[m * CACHELINE:]])
    return out.reshape(shape)


# speedup vs baseline: 23.9409x; 1.0882x over previous
import functools
import jax
import jax.numpy as jnp
from jax import lax
from jax.experimental import pallas as pl
from jax.experimental.pallas import tpu as pltpu
from jax.experimental.pallas import tpu_sc as plsc

CACHELINE = 64
THRESHOLD = 0.05
_NC = 2
_NS = 16
_NW = _NC * _NS
_L = 16
GROUP_ELEMS = CACHELINE * _L


def _make_cluster_call(num_groups: int):
    groups_per_worker = num_groups // _NW
    pairs_per_worker = groups_per_worker // 2
    mesh = plsc.VectorSubcoreMesh(core_axis_name="c", subcore_axis_name="s")

    @functools.partial(
        pl.kernel,
        out_type=jax.ShapeDtypeStruct((num_groups * GROUP_ELEMS,), jnp.float32),
        mesh=mesh,
        scratch_types=[
            pltpu.VMEM((GROUP_ELEMS,), jnp.float32),
            pltpu.VMEM((GROUP_ELEMS,), jnp.float32),
            pltpu.VMEM((GROUP_ELEMS,), jnp.float32),
            pltpu.VMEM((GROUP_ELEMS,), jnp.float32),
        ],
    )
    def cluster(x_hbm, out_hbm, xa, ba, xc, bc):
        wid = lax.axis_index("s") * _NC + lax.axis_index("c")

        def pair_body(p, carry):
            basea = (wid * groups_per_worker + 2 * p) * GROUP_ELEMS
            baseb = basea + GROUP_ELEMS
            pltpu.sync_copy(x_hbm.at[pl.ds(basea, GROUP_ELEMS)], xa)
            pltpu.sync_copy(x_hbm.at[pl.ds(baseb, GROUP_ELEMS)], xc)

            inf_row = jnp.full((_L,), jnp.inf, jnp.float32)

            def init_body(r, c):
                ba[pl.ds(r * _L, _L)] = inf_row
                bc[pl.ds(r * _L, _L)] = inf_row
                return c

            lax.fori_loop(0, CACHELINE - 1, init_body, 0)
            ba[pl.ds((CACHELINE - 1) * _L, _L)] = xa[pl.ds(0, _L)]
            bc[pl.ds((CACHELINE - 1) * _L, _L)] = xc[pl.ds(0, _L)]

            def j_body(j, carry2):
                xja = xa[pl.ds(j * _L, _L)]
                xjb = xc[pl.ds(j * _L, _L)]
                s4 = ((CACHELINE - j) // 4) * 4
                n4 = (CACHELINE - s4) // 4

                def k_body(t, kc):
                    ra, rb = kc
                    rowb = (s4 + t * 4) * _L
                    for u in range(4):
                        bva = ba[pl.ds(rowb + u * _L, _L)]
                        bvb = bc[pl.ds(rowb + u * _L, _L)]
                        ra = jnp.where(jnp.abs(bva - xja) < THRESHOLD, bva, ra)
                        rb = jnp.where(jnp.abs(bvb - xjb) < THRESHOLD, bvb, rb)
                    return ra, rb

                ra, rb = lax.fori_loop(0, n4, k_body, (xja, xjb))
                ba[pl.ds((CACHELINE - 1 - j) * _L, _L)] = jnp.where(
                    ra != xja, jnp.inf, xja
                )
                bc[pl.ds((CACHELINE - 1 - j) * _L, _L)] = jnp.where(
                    rb != xjb, jnp.inf, xjb
                )
                xa[pl.ds(j * _L, _L)] = ra
                xc[pl.ds(j * _L, _L)] = rb
                return carry2

            lax.fori_loop(1, CACHELINE, j_body, 0)
            pltpu.sync_copy(xa, out_hbm.at[pl.ds(basea, GROUP_ELEMS)])
            pltpu.sync_copy(xc, out_hbm.at[pl.ds(baseb, GROUP_ELEMS)])
            return carry

        lax.fori_loop(0, pairs_per_worker, pair_body, 0)

    return cluster


def kernel(x):
    shape = x.shape
    flat = x.reshape(-1)
    n = flat.shape[0]
    m = n // CACHELINE
    body = flat[: m * CACHELINE]

    num_groups = -(-m // _L)
    total_groups = num_groups + ((-num_groups) % (2 * _NW))
    pad_elems = total_groups * GROUP_ELEMS - m * CACHELINE
    arr = body
    if pad_elems:
        arr = jnp.concatenate([arr, jnp.zeros((pad_elems,), jnp.float32)])
    arr = arr.reshape(total_groups, _L, CACHELINE).transpose(0, 2, 1)
    arr = arr.reshape(-1)

    call = _make_cluster_call(total_groups)
    out = call(arr).reshape(total_groups, CACHELINE, _L).transpose(0, 2, 1)
    out = out.reshape(-1)[: m * CACHELINE]

    if m * CACHELINE != n:
        out = jnp.concatenate([out, flat[m * CACHELINE:]])
    return out.reshape(shape)
